# Initial kernel scaffold; baseline (speedup 1.0000x reference)
import jax
import jax.numpy as jnp
from jax.experimental import pallas as pl


def kernel(points, params):
    raise NotImplementedError("write your pallas kernel here")


# speedup vs baseline: 3.1769x; 3.1769x over previous
import functools
import jax
import jax.numpy as jnp
from jax import lax
from jax.experimental import pallas as pl
from jax.experimental.pallas import tpu as pltpu

KNN = 20
_RT = 256
_IDXPAD = 32


def _knn_kernel(x_ref, xx_ref, idx_ref):
    i = pl.program_id(1)
    rt = idx_ref.shape[0]
    C, N = x_ref.shape
    xb = x_ref[...].astype(jnp.bfloat16)
    xtb = x_ref[:, pl.ds(i * rt, rt)].astype(jnp.bfloat16)
    inner = -2.0 * jax.lax.dot_general(
        xtb, xb, (((0,), (0,)), ((), ())),
        preferred_element_type=jnp.float32)
    xi = xx_ref[0, pl.ds(i * rt, rt)].reshape(rt, 1)
    d = (-xx_ref[...] - inner) - xi
    iota = jax.lax.broadcasted_iota(jnp.int32, (rt, N), 1)
    for k in range(KNN):
        m = jnp.max(d, axis=1, keepdims=True)
        j = jnp.min(jnp.where(d == m, iota, N), axis=1, keepdims=True)
        idx_ref[:, k:k + 1] = j
        d = jnp.where(iota == j, -jnp.inf, d)


def _knn(x):
    B, C, N = x.shape
    xx = jnp.sum(x * x, axis=1, keepdims=True)
    grid = (B, N // _RT)
    idx = pl.pallas_call(
        _knn_kernel,
        grid=grid,
        in_specs=[
            pl.BlockSpec((None, C, N), lambda b, i: (b, 0, 0)),
            pl.BlockSpec((None, 1, N), lambda b, i: (b, 0, 0)),
        ],
        out_specs=pl.BlockSpec((None, _RT, _IDXPAD), lambda b, i: (b, i, 0)),
        out_shape=jax.ShapeDtypeStruct((B, N, _IDXPAD), jnp.int32),
    )(x, xx)
    return idx[:, :, :KNN]


def _get_graph_feature(x, idx):
    B, C, N = x.shape
    k = idx.shape[-1]
    fidx = (idx + jnp.arange(B).reshape(-1, 1, 1) * N).reshape(-1)
    xt = jnp.transpose(x, (0, 2, 1)).reshape(B * N, C)
    feat = xt[fidx].reshape(B, N, k, C)
    xr = jnp.broadcast_to(xt.reshape(B, N, 1, C), (B, N, k, C))
    feat = jnp.concatenate([feat - xr, xr], axis=3)
    return jnp.transpose(feat, (0, 3, 1, 2))


def _bn(x, g, b, axes):
    m = jnp.mean(x, axis=axes, keepdims=True)
    v = jnp.var(x, axis=axes, keepdims=True)
    sh = [1] * x.ndim
    sh[1] = g.shape[0]
    return (x - m) / jnp.sqrt(v + 1e-5) * g.reshape(sh) + b.reshape(sh)


def _lrelu(x):
    return jnp.where(x >= 0, x, 0.2 * x)


def _edgeconv(x, W, g, b):
    idx = _knn(x)
    f = _get_graph_feature(x, idx)
    f = _lrelu(_bn(jnp.einsum('oi,bink->bonk', W, f), g, b, (0, 2, 3)))
    return jnp.max(f, axis=-1)


def kernel(points, params):
    p = params
    B, N = points.shape[0], points.shape[1]
    x = points.reshape(B, -1, N)
    x1 = _edgeconv(x, p['W1'], p['g1'], p['b1'])
    x2 = _edgeconv(x1, p['W2'], p['g2'], p['b2'])
    x3 = _edgeconv(x2, p['W3'], p['g3'], p['b3'])
    x4 = _edgeconv(x3, p['W4'], p['g4'], p['b4'])
    xc = jnp.concatenate([x1, x2, x3, x4], axis=1)
    x5 = _lrelu(_bn(jnp.einsum('oi,bin->bon', p['W5'], xc), p['g5'], p['b5'], (0, 2)))
    h = jnp.concatenate([jnp.max(x5, axis=-1), jnp.mean(x5, axis=-1)], axis=1)
    h = _lrelu(_bn(h @ p['Wl1'].T, p['g6'], p['b6'], (0,)))
    h = _lrelu(_bn(h @ p['Wl2'].T + p['bl2'], p['g7'], p['b7'], (0,)))
    return h @ p['Wl3'].T + p['bl3']


# speedup vs baseline: 3.3424x; 1.0521x over previous
import functools
import jax
import jax.numpy as jnp
from jax import lax
from jax.experimental import pallas as pl
from jax.experimental.pallas import tpu as pltpu

KNN = 20
_RT = 256
_IDXPAD = 32


def _knn_kernel(x_ref, xx_ref, idx_ref):
    i = pl.program_id(1)
    rt = idx_ref.shape[0]
    C, N = x_ref.shape
    xb = x_ref[...].astype(jnp.bfloat16)
    xtb = x_ref[:, pl.ds(i * rt, rt)].astype(jnp.bfloat16)
    inner = -2.0 * jax.lax.dot_general(
        xtb, xb, (((0,), (0,)), ((), ())),
        preferred_element_type=jnp.float32)
    xi = xx_ref[0, pl.ds(i * rt, rt)].reshape(rt, 1)
    d = (-xx_ref[...] - inner) - xi
    iota = jax.lax.broadcasted_iota(jnp.int32, (rt, N), 1)
    for k in range(KNN):
        m = jnp.max(d, axis=1, keepdims=True)
        j = jnp.min(jnp.where(d == m, iota, N), axis=1, keepdims=True)
        idx_ref[:, k:k + 1] = j
        d = jnp.where(iota == j, -jnp.inf, d)


def _knn(x):
    B, C, N = x.shape
    xx = jnp.sum(x * x, axis=1, keepdims=True)
    grid = (B, N // _RT)
    idx = pl.pallas_call(
        _knn_kernel,
        grid=grid,
        in_specs=[
            pl.BlockSpec((None, C, N), lambda b, i: (b, 0, 0)),
            pl.BlockSpec((None, 1, N), lambda b, i: (b, 0, 0)),
        ],
        out_specs=pl.BlockSpec((None, _RT, _IDXPAD), lambda b, i: (b, i, 0)),
        out_shape=jax.ShapeDtypeStruct((B, N, _IDXPAD), jnp.int32),
    )(x, xx)
    return idx[:, :, :KNN]


_CT = 256


def _conv_kernel(xg_ref, x_ref, w_ref, m_ref, s1_ref, s2_ref):
    xi = x_ref[...]
    xib = xi.astype(jnp.bfloat16)
    wb = w_ref[...].astype(jnp.bfloat16)
    macc = None
    ts = []
    for k in range(KNN):
        diff = (xg_ref[k] - xi).astype(jnp.bfloat16)
        f = jnp.concatenate([diff, xib], axis=0)
        t = jax.lax.dot_general(wb, f, (((1,), (0,)), ((), ())),
                                preferred_element_type=jnp.float32)
        macc = t if macc is None else jnp.maximum(macc, t)
        ts.append(t)

    def _tree(vals):
        while len(vals) > 1:
            nxt = [vals[i] + vals[i + 1] for i in range(0, len(vals) - 1, 2)]
            if len(vals) % 2:
                nxt.append(vals[-1])
            vals = nxt
        return vals[0]

    m_ref[...] = macc
    s1_ref[...] = jnp.sum(_tree(list(ts)), axis=1, keepdims=True)
    s2_ref[...] = jnp.sum(_tree([t * t for t in ts]), axis=1, keepdims=True)


def _edgeconv_fused(x, idx, W, g, b):
    B, C, N = x.shape
    O = W.shape[0]
    fidx = (idx + jnp.arange(B).reshape(-1, 1, 1) * N).reshape(-1)
    xt = jnp.transpose(x, (0, 2, 1)).reshape(B * N, C)
    xg = xt[fidx].reshape(B, N, KNN, C)
    xg = jnp.transpose(xg, (0, 2, 3, 1))
    nt = N // _CT
    grid = (B, nt)
    m, s1, s2 = pl.pallas_call(
        _conv_kernel,
        grid=grid,
        in_specs=[
            pl.BlockSpec((None, KNN, C, _CT), lambda bb, i: (bb, 0, 0, i)),
            pl.BlockSpec((None, C, _CT), lambda bb, i: (bb, 0, i)),
            pl.BlockSpec((O, 2 * C), lambda bb, i: (0, 0)),
        ],
        out_specs=[
            pl.BlockSpec((None, O, _CT), lambda bb, i: (bb, 0, i)),
            pl.BlockSpec((None, None, O, 1), lambda bb, i: (bb, i, 0, 0)),
            pl.BlockSpec((None, None, O, 1), lambda bb, i: (bb, i, 0, 0)),
        ],
        out_shape=[
            jax.ShapeDtypeStruct((B, O, N), jnp.float32),
            jax.ShapeDtypeStruct((B, nt, O, 1), jnp.float32),
            jax.ShapeDtypeStruct((B, nt, O, 1), jnp.float32),
        ],
    )(xg, x, W)
    cnt = B * N * KNN
    mean = jnp.sum(s1, axis=(0, 1, 3)) / cnt
    var = jnp.sum(s2, axis=(0, 1, 3)) / cnt - mean * mean
    y = (m - mean.reshape(1, O, 1)) / jnp.sqrt(var + 1e-5).reshape(1, O, 1)
    y = y * g.reshape(1, O, 1) + b.reshape(1, O, 1)
    return jnp.where(y >= 0, y, 0.2 * y)


def _bn(x, g, b, axes):
    m = jnp.mean(x, axis=axes, keepdims=True)
    v = jnp.var(x, axis=axes, keepdims=True)
    sh = [1] * x.ndim
    sh[1] = g.shape[0]
    return (x - m) / jnp.sqrt(v + 1e-5) * g.reshape(sh) + b.reshape(sh)


def _lrelu(x):
    return jnp.where(x >= 0, x, 0.2 * x)


def _edgeconv_exact(x, W, g, b):
    B, C, N = x.shape
    idx = _knn(x)
    fidx = (idx + jnp.arange(B).reshape(-1, 1, 1) * N).reshape(-1)
    xt = jnp.transpose(x, (0, 2, 1)).reshape(B * N, C)
    feat = xt[fidx].reshape(B, N, KNN, C)
    xr = jnp.broadcast_to(xt.reshape(B, N, 1, C), (B, N, KNN, C))
    feat = jnp.concatenate([feat - xr, xr], axis=3)
    f = jnp.transpose(feat, (0, 3, 1, 2))
    f = _lrelu(_bn(jnp.einsum('oi,bink->bonk', W, f), g, b, (0, 2, 3)))
    return jnp.max(f, axis=-1)


def _edgeconv(x, W, g, b):
    idx = _knn(x)
    return _edgeconv_fused(x, idx, W, g, b)


def kernel(points, params):
    p = params
    B, N = points.shape[0], points.shape[1]
    x = points.reshape(B, -1, N)
    x1 = _edgeconv_exact(x, p['W1'], p['g1'], p['b1'])
    x2 = _edgeconv_exact(x1, p['W2'], p['g2'], p['b2'])
    x3 = _edgeconv(x2, p['W3'], p['g3'], p['b3'])
    x4 = _edgeconv(x3, p['W4'], p['g4'], p['b4'])
    xc = jnp.concatenate([x1, x2, x3, x4], axis=1)
    x5 = _lrelu(_bn(jnp.einsum('oi,bin->bon', p['W5'], xc), p['g5'], p['b5'], (0, 2)))
    h = jnp.concatenate([jnp.max(x5, axis=-1), jnp.mean(x5, axis=-1)], axis=1)
    h = _lrelu(_bn(h @ p['Wl1'].T, p['g6'], p['b6'], (0,)))
    h = _lrelu(_bn(h @ p['Wl2'].T + p['bl2'], p['g7'], p['b7'], (0,)))
    return h @ p['Wl3'].T + p['bl3']
